# Initial kernel scaffold; baseline (speedup 1.0000x reference)
import jax
import jax.numpy as jnp
from jax.experimental import pallas as pl


def kernel(logits):
    raise NotImplementedError("write your pallas kernel here")


# speedup vs baseline: 23.8288x; 23.8288x over previous
import jax
import jax.numpy as jnp
from jax import lax
from jax.experimental import pallas as pl

_B, _BW, _BE, _T, _V = 8, 4, 8, 16, 100000
_PAD, _BOS, _EOS = 0, 1, 2
_SUB = 8
_NEG = float("-inf")


def _topk_body(bw, t, v, be, sub):
    w = v // sub

    def body(x_ref, vals_ref, idxs_ref):
        x = x_ref[0]
        m = jnp.max(x)
        lse = m + jnp.log(jnp.sum(jnp.exp(x - m)))
        r_i = lax.broadcasted_iota(jnp.int32, (sub, w), 0)
        c_i = lax.broadcasted_iota(jnp.int32, (sub, w), 1)
        oidx = r_i * w + c_i
        ke = lax.broadcasted_iota(jnp.int32, (1, 1, be), 2)
        vout = jnp.zeros((1, 1, be), jnp.float32)
        iout = jnp.zeros((1, 1, be), jnp.int32)
        cur = x
        for k in range(be):
            mk = jnp.max(cur)
            ik = jnp.min(jnp.where(cur == mk, oidx, v))
            vout = jnp.where(ke == k, mk - lse, vout)
            iout = jnp.where(ke == k, ik, iout)
            if k + 1 < be:
                cur = jnp.where(oidx == ik, _NEG, cur)
        vals_ref[...] = vout
        idxs_ref[...] = iout

    return body


def _topk_call(logits, b, bw, t, v, be, sub, interpret=False):
    rows = b * bw * t
    w = v // sub
    x = logits.reshape(rows, sub, w)
    vals, idxs = pl.pallas_call(
        _topk_body(bw, t, v, be, sub),
        grid=(rows,),
        in_specs=[pl.BlockSpec((1, sub, w), lambda i: (i, 0, 0))],
        out_specs=[
            pl.BlockSpec((1, 1, be), lambda i: (i, 0, 0)),
            pl.BlockSpec((1, 1, be), lambda i: (i, 0, 0)),
        ],
        out_shape=[
            jax.ShapeDtypeStruct((rows, 1, be), jnp.float32),
            jax.ShapeDtypeStruct((rows, 1, be), jnp.int32),
        ],
        interpret=interpret,
    )(x)
    return vals, idxs


def _beam_body(b, bw, be, t_steps):
    ncand = bw * be
    slen = t_steps + 1

    def _sel(cond_arrays, values):
        out = values[-1]
        for c, val in zip(cond_arrays[:-1][::-1], values[:-1][::-1]):
            out = jnp.where(c, val, out)
        return out

    def body(vals_ref, idxs_ref, seq_ref, score_ref, len_ref):
        lane = lax.broadcasted_iota(jnp.int32, (b, ncand), 1)
        w_lane = lane // be
        e_lane = lane % be
        col = lax.broadcasted_iota(jnp.int32, (b, slen), 1)
        bw_iota = lax.broadcasted_iota(jnp.int32, (b, bw), 1)

        v0 = vals_ref[0]
        i0 = idxs_ref[0]
        seqs, rs, nlens, lasts = [], [], [], []
        for wdx in range(bw):
            tok = i0[:, wdx:wdx + 1]
            val = v0[:, wdx:wdx + 1]
            seq = jnp.where(col == 0, _BOS, jnp.where(col == 1, tok, _PAD))
            seqs.append(seq.astype(jnp.int32))
            rs.append(val)
            nlens.append(1 + jnp.where(tok != _PAD, 1, 0))
            lasts.append(tok)

        for t in range(1, t_steps):
            v_t = vals_ref[t]
            i_t = idxs_ref[t]
            dones = [jnp.where((l == _PAD) | (l == _EOS), 1, 0) for l in lasts]
            wconds = [w_lane == wdx for wdx in range(bw)]
            done_f = _sel(wconds, [jnp.broadcast_to(d, (b, ncand)) for d in dones])
            r_f = _sel(wconds, [jnp.broadcast_to(r, (b, ncand)) for r in rs])
            n_f = _sel(wconds, [jnp.broadcast_to(n, (b, ncand)) for n in nlens])
            cv = jnp.where(done_f != 0, jnp.where(e_lane == 0, 0.0, _NEG), v_t)
            ct = jnp.where(done_f != 0, e_lane, i_t)
            cand_r = r_f + cv
            cand_n = n_f + jnp.where(ct != _PAD, 1, 0)
            lp = (5.0 + cand_n.astype(jnp.float32)) / 6.0
            bs = cand_r / lp

            new_seqs, new_rs, new_nlens, new_lasts = [], [], [], []
            for _k in range(bw):
                mk = jnp.max(bs, axis=1, keepdims=True)
                pick = jnp.min(jnp.where(bs == mk, lane, ncand), axis=1,
                               keepdims=True)
                bs = jnp.where(lane == pick, _NEG, bs)
                hit = lane == pick
                tok = jnp.sum(jnp.where(hit, ct, 0), axis=1, keepdims=True)
                rk = jnp.sum(jnp.where(hit, cand_r, 0.0), axis=1, keepdims=True)
                nk = jnp.sum(jnp.where(hit, cand_n, 0), axis=1, keepdims=True)
                w_pick = pick // be
                wc = [w_pick == wdx for wdx in range(bw)]
                src_seq = _sel(wc, seqs)
                new_seqs.append(jnp.where(col == t + 1, tok, src_seq))
                new_rs.append(rk)
                new_nlens.append(nk)
                new_lasts.append(tok)
            seqs, rs, nlens, lasts = new_seqs, new_rs, new_nlens, new_lasts

        bsf = jnp.concatenate(
            [r / ((5.0 + n.astype(jnp.float32)) / 6.0)
             for r, n in zip(rs, nlens)], axis=1)
        nlen_all = jnp.concatenate(nlens, axis=1)
        scores_out = jnp.zeros((b, bw), jnp.float32)
        cur = bsf
        for k in range(bw):
            mk = jnp.max(cur, axis=1, keepdims=True)
            pick = jnp.min(jnp.where(cur == mk, bw_iota, bw), axis=1,
                           keepdims=True)
            cur = jnp.where(bw_iota == pick, _NEG, cur)
            scores_out = jnp.where(bw_iota == k, mk, scores_out)
            wc = [pick == wdx for wdx in range(bw)]
            seq_ref[:, k, :] = _sel(wc, seqs)
        score_ref[...] = scores_out
        len_ref[...] = nlen_all

    return body


def _beam_call(vals, idxs, b, bw, be, t_steps, interpret=False):
    ncand = bw * be
    slen = t_steps + 1
    seq, scores, lens = pl.pallas_call(
        _beam_body(b, bw, be, t_steps),
        out_shape=[
            jax.ShapeDtypeStruct((b, bw, slen), jnp.int32),
            jax.ShapeDtypeStruct((b, bw), jnp.float32),
            jax.ShapeDtypeStruct((b, bw), jnp.int32),
        ],
        interpret=interpret,
    )(vals, idxs)
    return seq, scores, lens


def _run(logits, b, bw, be, t_steps, v, sub, interpret=False):
    vals, idxs = _topk_call(logits, b, bw, t_steps, v, be, sub, interpret)
    vals = vals.reshape(b, bw, t_steps, be).transpose(2, 0, 1, 3)
    vals = vals.reshape(t_steps, b, bw * be)
    idxs = idxs.reshape(b, bw, t_steps, be).transpose(2, 0, 1, 3)
    idxs = idxs.reshape(t_steps, b, bw * be)
    return _beam_call(vals, idxs, b, bw, be, t_steps, interpret)


def kernel(logits):
    return _run(logits, _B, _BW, _BE, _T, _V, _SUB)


# speedup vs baseline: 71.9103x; 3.0178x over previous
import jax
import jax.numpy as jnp
from jax import lax
from jax.experimental import pallas as pl

_B, _BW, _BE, _T, _V = 8, 4, 8, 16, 100000
_PAD, _BOS, _EOS = 0, 1, 2
_SUB = 8
_NEG = float("-inf")


def _topk_body(bw, t, v, be, sub, rpb):
    w = v // sub

    def body(x_ref, vals_ref, idxs_ref):
        r_i = lax.broadcasted_iota(jnp.int32, (1, sub, w), 1)
        c_i = lax.broadcasted_iota(jnp.int32, (1, sub, w), 2)
        oidx = r_i * w + c_i
        ke = lax.broadcasted_iota(jnp.int32, (rpb, 1, be), 2)

        def red(a, fn):
            return fn(fn(a, 2), 1)[:, None, None]

        x = x_ref[...]
        m = red(x, lambda a, ax: jnp.max(a, axis=ax))
        s = red(jnp.exp(x - m), lambda a, ax: jnp.sum(a, axis=ax))
        lse = m + jnp.log(s)
        vout = jnp.zeros((rpb, 1, be), jnp.float32)
        iout = jnp.zeros((rpb, 1, be), jnp.int32)
        cur = x
        for k in range(be):
            mk = red(cur, lambda a, ax: jnp.max(a, axis=ax))
            ik = red(jnp.where(cur == mk, oidx, v),
                     lambda a, ax: jnp.min(a, axis=ax))
            vout = jnp.where(ke == k, mk - lse, vout)
            iout = jnp.where(ke == k, ik, iout)
            if k + 1 < be:
                cur = jnp.where(oidx == ik, _NEG, cur)
        vals_ref[...] = vout
        idxs_ref[...] = iout

    return body


def _topk_call(logits, b, bw, t, v, be, sub, rpb, interpret=False):
    rows = b * bw * t
    w = v // sub
    x = logits.reshape(rows, sub, w)
    vals, idxs = pl.pallas_call(
        _topk_body(bw, t, v, be, sub, rpb),
        grid=(rows // rpb,),
        in_specs=[pl.BlockSpec((rpb, sub, w), lambda i: (i, 0, 0))],
        out_specs=[
            pl.BlockSpec((rpb, 1, be), lambda i: (i, 0, 0)),
            pl.BlockSpec((rpb, 1, be), lambda i: (i, 0, 0)),
        ],
        out_shape=[
            jax.ShapeDtypeStruct((rows, 1, be), jnp.float32),
            jax.ShapeDtypeStruct((rows, 1, be), jnp.int32),
        ],
        interpret=interpret,
    )(x)
    return vals, idxs


def _beam_body(b, bw, be, t_steps):
    ncand = bw * be
    slen = t_steps + 1

    def _sel(cond_arrays, values):
        out = values[-1]
        for c, val in zip(cond_arrays[:-1][::-1], values[:-1][::-1]):
            out = jnp.where(c, val, out)
        return out

    def body(vals_ref, idxs_ref, seq_ref, score_ref, len_ref):
        lane = lax.broadcasted_iota(jnp.int32, (b, ncand), 1)
        w_lane = lane // be
        e_lane = lane % be
        col = lax.broadcasted_iota(jnp.int32, (b, slen), 1)
        bw_iota = lax.broadcasted_iota(jnp.int32, (b, bw), 1)

        v0 = vals_ref[0]
        i0 = idxs_ref[0]
        seqs, rs, nlens, lasts = [], [], [], []
        for wdx in range(bw):
            tok = i0[:, wdx:wdx + 1]
            val = v0[:, wdx:wdx + 1]
            seq = jnp.where(col == 0, _BOS, jnp.where(col == 1, tok, _PAD))
            seqs.append(seq.astype(jnp.int32))
            rs.append(val)
            nlens.append(1 + jnp.where(tok != _PAD, 1, 0))
            lasts.append(tok)

        for t in range(1, t_steps):
            v_t = vals_ref[t]
            i_t = idxs_ref[t]
            dones = [jnp.where((l == _PAD) | (l == _EOS), 1, 0) for l in lasts]
            wconds = [w_lane == wdx for wdx in range(bw)]
            done_f = _sel(wconds, [jnp.broadcast_to(d, (b, ncand)) for d in dones])
            r_f = _sel(wconds, [jnp.broadcast_to(r, (b, ncand)) for r in rs])
            n_f = _sel(wconds, [jnp.broadcast_to(n, (b, ncand)) for n in nlens])
            cv = jnp.where(done_f != 0, jnp.where(e_lane == 0, 0.0, _NEG), v_t)
            ct = jnp.where(done_f != 0, e_lane, i_t)
            cand_r = r_f + cv
            cand_n = n_f + jnp.where(ct != _PAD, 1, 0)
            lp = (5.0 + cand_n.astype(jnp.float32)) / 6.0
            bs = cand_r / lp

            new_seqs, new_rs, new_nlens, new_lasts = [], [], [], []
            for _k in range(bw):
                mk = jnp.max(bs, axis=1, keepdims=True)
                pick = jnp.min(jnp.where(bs == mk, lane, ncand), axis=1,
                               keepdims=True)
                bs = jnp.where(lane == pick, _NEG, bs)
                hit = lane == pick
                tok = jnp.sum(jnp.where(hit, ct, 0), axis=1, keepdims=True)
                rk = jnp.sum(jnp.where(hit, cand_r, 0.0), axis=1, keepdims=True)
                nk = jnp.sum(jnp.where(hit, cand_n, 0), axis=1, keepdims=True)
                w_pick = pick // be
                wc = [w_pick == wdx for wdx in range(bw)]
                src_seq = _sel(wc, seqs)
                new_seqs.append(jnp.where(col == t + 1, tok, src_seq))
                new_rs.append(rk)
                new_nlens.append(nk)
                new_lasts.append(tok)
            seqs, rs, nlens, lasts = new_seqs, new_rs, new_nlens, new_lasts

        bsf = jnp.concatenate(
            [r / ((5.0 + n.astype(jnp.float32)) / 6.0)
             for r, n in zip(rs, nlens)], axis=1)
        nlen_all = jnp.concatenate(nlens, axis=1)
        scores_out = jnp.zeros((b, bw), jnp.float32)
        cur = bsf
        for k in range(bw):
            mk = jnp.max(cur, axis=1, keepdims=True)
            pick = jnp.min(jnp.where(cur == mk, bw_iota, bw), axis=1,
                           keepdims=True)
            cur = jnp.where(bw_iota == pick, _NEG, cur)
            scores_out = jnp.where(bw_iota == k, mk, scores_out)
            wc = [pick == wdx for wdx in range(bw)]
            seq_ref[:, k, :] = _sel(wc, seqs)
        score_ref[...] = scores_out
        len_ref[...] = nlen_all

    return body


def _beam_call(vals, idxs, b, bw, be, t_steps, interpret=False):
    ncand = bw * be
    slen = t_steps + 1
    seq, scores, lens = pl.pallas_call(
        _beam_body(b, bw, be, t_steps),
        out_shape=[
            jax.ShapeDtypeStruct((b, bw, slen), jnp.int32),
            jax.ShapeDtypeStruct((b, bw), jnp.float32),
            jax.ShapeDtypeStruct((b, bw), jnp.int32),
        ],
        interpret=interpret,
    )(vals, idxs)
    return seq, scores, lens


def _run(logits, b, bw, be, t_steps, v, sub, rpb, interpret=False):
    vals, idxs = _topk_call(logits, b, bw, t_steps, v, be, sub, rpb, interpret)
    vals = vals.reshape(b, bw, t_steps, be).transpose(2, 0, 1, 3)
    vals = vals.reshape(t_steps, b, bw * be)
    idxs = idxs.reshape(b, bw, t_steps, be).transpose(2, 0, 1, 3)
    idxs = idxs.reshape(t_steps, b, bw * be)
    return _beam_call(vals, idxs, b, bw, be, t_steps, interpret)


def kernel(logits):
    return _run(logits, _B, _BW, _BE, _T, _V, _SUB, 16)


# speedup vs baseline: 79.3150x; 1.1030x over previous
import jax
import jax.numpy as jnp
from jax import lax
from jax.experimental import pallas as pl

_B, _BW, _BE, _T, _V = 8, 4, 8, 16, 100000
_PAD, _BOS, _EOS = 0, 1, 2
_SUB = 8
_NEG = float("-inf")


def _topk_body(bw, t, v, be, sub, rpb):
    w = v // sub

    def body(x_ref, vals_ref, idxs_ref):
        r_i = lax.broadcasted_iota(jnp.int32, (1, sub, w), 1)
        c_i = lax.broadcasted_iota(jnp.int32, (1, sub, w), 2)
        oidx = r_i * w + c_i
        ke = lax.broadcasted_iota(jnp.int32, (rpb, 1, be), 2)

        def red(a, fn):
            return fn(fn(a, 2), 1)[:, None, None]

        x = x_ref[...]
        m = red(x, lambda a, ax: jnp.max(a, axis=ax))
        s = red(jnp.exp(x - m), lambda a, ax: jnp.sum(a, axis=ax))
        lse = m + jnp.log(s)
        vout = jnp.zeros((rpb, 1, be), jnp.float32)
        iout = jnp.zeros((rpb, 1, be), jnp.int32)
        cur = x
        for k in range(be):
            mk = red(cur, lambda a, ax: jnp.max(a, axis=ax))
            ik = red(jnp.where(cur == mk, oidx, v),
                     lambda a, ax: jnp.min(a, axis=ax))
            vout = jnp.where(ke == k, mk - lse, vout)
            iout = jnp.where(ke == k, ik, iout)
            if k + 1 < be:
                cur = jnp.where(oidx == ik, _NEG, cur)
        vals_ref[...] = vout
        idxs_ref[...] = iout

    return body


def _topk_call(logits, b, bw, t, v, be, sub, rpb, interpret=False):
    rows = b * bw * t
    w = v // sub
    x = logits.reshape(rows, sub, w)
    vals, idxs = pl.pallas_call(
        _topk_body(bw, t, v, be, sub, rpb),
        grid=(rows // rpb,),
        in_specs=[pl.BlockSpec((rpb, sub, w), lambda i: (i, 0, 0))],
        out_specs=[
            pl.BlockSpec((rpb, 1, be), lambda i: (i, 0, 0)),
            pl.BlockSpec((rpb, 1, be), lambda i: (i, 0, 0)),
        ],
        out_shape=[
            jax.ShapeDtypeStruct((rows, 1, be), jnp.float32),
            jax.ShapeDtypeStruct((rows, 1, be), jnp.int32),
        ],
        interpret=interpret,
    )(x)
    return vals, idxs


def _beam_body(b, bw, be, t_steps):
    ncand = bw * be
    slen = t_steps + 1

    def _sel(cond_arrays, values):
        out = values[-1]
        for c, val in zip(cond_arrays[:-1][::-1], values[:-1][::-1]):
            out = jnp.where(c, val, out)
        return out

    def body(vals_ref, idxs_ref, seq_ref, score_ref, len_ref):
        lane = lax.broadcasted_iota(jnp.int32, (b, ncand), 1)
        w_lane = lane // be
        e_lane = lane % be
        col = lax.broadcasted_iota(jnp.int32, (b, slen), 1)
        bw_iota = lax.broadcasted_iota(jnp.int32, (b, bw), 1)

        v0 = vals_ref[0]
        i0 = idxs_ref[0]
        seqs, rs, nlens, lasts = [], [], [], []
        for wdx in range(bw):
            tok = i0[:, wdx:wdx + 1]
            val = v0[:, wdx:wdx + 1]
            seq = jnp.where(col == 0, _BOS, jnp.where(col == 1, tok, _PAD))
            seqs.append(seq.astype(jnp.int32))
            rs.append(val)
            nlens.append(1 + jnp.where(tok != _PAD, 1, 0))
            lasts.append(tok)

        for t in range(1, t_steps):
            v_t = vals_ref[t]
            i_t = idxs_ref[t]
            dones = [jnp.where((l == _PAD) | (l == _EOS), 1, 0) for l in lasts]
            wconds = [w_lane == wdx for wdx in range(bw)]
            done_f = _sel(wconds, [jnp.broadcast_to(d, (b, ncand)) for d in dones])
            r_f = _sel(wconds, [jnp.broadcast_to(r, (b, ncand)) for r in rs])
            n_f = _sel(wconds, [jnp.broadcast_to(n, (b, ncand)) for n in nlens])
            cv = jnp.where(done_f != 0, jnp.where(e_lane == 0, 0.0, _NEG), v_t)
            ct = jnp.where(done_f != 0, e_lane, i_t)
            cand_r = r_f + cv
            cand_n = n_f + jnp.where(ct != _PAD, 1, 0)
            lp = (5.0 + cand_n.astype(jnp.float32)) / 6.0
            bs = cand_r / lp

            new_seqs, new_rs, new_nlens, new_lasts = [], [], [], []
            for _k in range(bw):
                mk = jnp.max(bs, axis=1, keepdims=True)
                pick = jnp.min(jnp.where(bs == mk, lane, ncand), axis=1,
                               keepdims=True)
                bs = jnp.where(lane == pick, _NEG, bs)
                hit = lane == pick
                tok = jnp.sum(jnp.where(hit, ct, 0), axis=1, keepdims=True)
                rk = jnp.sum(jnp.where(hit, cand_r, 0.0), axis=1, keepdims=True)
                nk = jnp.sum(jnp.where(hit, cand_n, 0), axis=1, keepdims=True)
                w_pick = pick // be
                wc = [w_pick == wdx for wdx in range(bw)]
                src_seq = _sel(wc, seqs)
                new_seqs.append(jnp.where(col == t + 1, tok, src_seq))
                new_rs.append(rk)
                new_nlens.append(nk)
                new_lasts.append(tok)
            seqs, rs, nlens, lasts = new_seqs, new_rs, new_nlens, new_lasts

        bsf = jnp.concatenate(
            [r / ((5.0 + n.astype(jnp.float32)) / 6.0)
             for r, n in zip(rs, nlens)], axis=1)
        nlen_all = jnp.concatenate(nlens, axis=1)
        scores_out = jnp.zeros((b, bw), jnp.float32)
        cur = bsf
        for k in range(bw):
            mk = jnp.max(cur, axis=1, keepdims=True)
            pick = jnp.min(jnp.where(cur == mk, bw_iota, bw), axis=1,
                           keepdims=True)
            cur = jnp.where(bw_iota == pick, _NEG, cur)
            scores_out = jnp.where(bw_iota == k, mk, scores_out)
            wc = [pick == wdx for wdx in range(bw)]
            seq_ref[:, k, :] = _sel(wc, seqs)
        score_ref[...] = scores_out
        len_ref[...] = nlen_all

    return body


def _beam_call(vals, idxs, b, bw, be, t_steps, interpret=False):
    ncand = bw * be
    slen = t_steps + 1
    seq, scores, lens = pl.pallas_call(
        _beam_body(b, bw, be, t_steps),
        out_shape=[
            jax.ShapeDtypeStruct((b, bw, slen), jnp.int32),
            jax.ShapeDtypeStruct((b, bw), jnp.float32),
            jax.ShapeDtypeStruct((b, bw), jnp.int32),
        ],
        interpret=interpret,
    )(vals, idxs)
    return seq, scores, lens


def _run(logits, b, bw, be, t_steps, v, sub, rpb, interpret=False):
    vals, idxs = _topk_call(logits, b, bw, t_steps, v, be, sub, rpb, interpret)
    vals = vals.reshape(b, bw, t_steps, be).transpose(2, 0, 1, 3)
    vals = vals.reshape(t_steps, b, bw * be)
    idxs = idxs.reshape(b, bw, t_steps, be).transpose(2, 0, 1, 3)
    idxs = idxs.reshape(t_steps, b, bw * be)
    return _beam_call(vals, idxs, b, bw, be, t_steps, interpret)


def kernel(logits):
    return _run(logits, _B, _BW, _BE, _T, _V, _SUB, 32)
